# baseline (device time: 47133 ns/iter reference)
import jax
import jax.numpy as jnp
from jax import lax
from jax.experimental import pallas as pl
from jax.experimental.pallas import tpu as pltpu

N_DEV = 4
N_SUB = 2


def kernel(x, w_mat):
    m_per, k = x.shape
    _, n_per = w_mat.shape
    m_half = m_per // 2
    m_sub = m_half // N_SUB

    def body(x_hbm, w_hbm, out_hbm,
             x_vmem, w_vmem, w_bf, comm_a, comm_b, stage,
             x_sems, w_sem, out_sems,
             send_a, recv_a, send_b, recv_b):
        my_pos = lax.axis_index("i")
        left = lax.rem(my_pos + N_DEV - 1, N_DEV)
        right = lax.rem(my_pos + 1, N_DEV)

        def sub(s):
            return slice(s * m_sub, (s + 1) * m_sub)

        def bsub(s):
            return slice(m_half + s * m_sub, m_half + (s + 1) * m_sub)

        xa_copies = []
        xb_copies = []
        for s in range(N_SUB):
            ca = pltpu.make_async_copy(
                x_hbm.at[sub(s)], x_vmem.at[sub(s)], x_sems.at[0, s])
            cb = pltpu.make_async_copy(
                x_hbm.at[bsub(s)], x_vmem.at[bsub(s)], x_sems.at[1, s])
            ca.start()
            cb.start()
            xa_copies.append(ca)
            xb_copies.append(cb)
        w_copy = pltpu.make_async_copy(w_hbm, w_vmem, w_sem)
        w_copy.start()

        barrier_sem = pltpu.get_barrier_semaphore()
        for nbr in [left, right]:
            pl.semaphore_signal(
                barrier_sem, inc=1,
                device_id=(nbr,), device_id_type=pl.DeviceIdType.MESH,
            )
        pl.semaphore_wait(barrier_sem, 2)

        rdma_a = {}
        rdma_b = {}

        def make_a(h, s):
            r = pltpu.make_async_remote_copy(
                src_ref=comm_a.at[h, sub(s)],
                dst_ref=comm_a.at[h + 1, sub(s)],
                send_sem=send_a.at[h, s],
                recv_sem=recv_a.at[h, s],
                device_id=(right,),
                device_id_type=pl.DeviceIdType.MESH,
            )
            rdma_a[(h, s)] = r
            return r

        def make_b(h, s):
            r = pltpu.make_async_remote_copy(
                src_ref=comm_b.at[h, sub(s)],
                dst_ref=comm_b.at[h + 1, sub(s)],
                send_sem=send_b.at[h, s],
                recv_sem=recv_b.at[h, s],
                device_id=(left,),
                device_id_type=pl.DeviceIdType.MESH,
            )
            rdma_b[(h, s)] = r
            return r

        for s in range(N_SUB):
            xa_copies[s].wait()
            comm_a[0, sub(s)] = x_vmem[sub(s)].astype(jnp.bfloat16)
            make_a(0, s).start()
            xb_copies[s].wait()
            comm_b[0, sub(s)] = x_vmem[bsub(s)].astype(jnp.bfloat16)
            make_b(0, s).start()

        w_copy.wait()
        w_bf[...] = w_vmem[...].astype(jnp.bfloat16)

        def silu_gemm(src):
            y = jnp.dot(src, w_bf[...], preferred_element_type=jnp.float32)
            return y * jax.nn.sigmoid(y)

        out_copies = []

        def store_piece(row0, val):
            kidx = len(out_copies)
            slot = kidx % 2
            if kidx >= 2:
                out_copies[kidx - 2].wait()
            stage[slot] = val
            c = pltpu.make_async_copy(
                stage.at[slot],
                out_hbm.at[pl.ds(row0, m_sub)],
                out_sems.at[kidx],
            )
            c.start()
            out_copies.append(c)

        for s in range(N_SUB):
            store_piece(my_pos * m_per + s * m_sub,
                        silu_gemm(comm_a[0, sub(s)]))
            store_piece(my_pos * m_per + m_half + s * m_sub,
                        silu_gemm(comm_b[0, sub(s)]))

        for h in range(1, N_DEV):
            o_a = lax.rem(my_pos - h + N_DEV, N_DEV)
            o_b = lax.rem(my_pos + h, N_DEV)
            for s in range(N_SUB):
                rdma_a[(h - 1, s)].wait_recv()
                if h < N_DEV - 1:
                    make_a(h, s).start()
                rdma_b[(h - 1, s)].wait_recv()
                if h < N_DEV - 1:
                    make_b(h, s).start()
                store_piece(o_a * m_per + s * m_sub,
                            silu_gemm(comm_a[h, sub(s)]))
                store_piece(o_b * m_per + m_half + s * m_sub,
                            silu_gemm(comm_b[h, sub(s)]))

        for c in out_copies[-2:]:
            c.wait()
        for d in (rdma_a, rdma_b):
            for r in d.values():
                r.wait_send()

    n_stores = 2 * N_SUB * N_DEV
    return pl.pallas_call(
        body,
        out_shape=jax.ShapeDtypeStruct((N_DEV * m_per, n_per), jnp.float32),
        in_specs=[
            pl.BlockSpec(memory_space=pltpu.MemorySpace.HBM),
            pl.BlockSpec(memory_space=pltpu.MemorySpace.HBM),
        ],
        out_specs=pl.BlockSpec(memory_space=pltpu.MemorySpace.HBM),
        scratch_shapes=[
            pltpu.VMEM((m_per, k), jnp.float32),
            pltpu.VMEM((k, n_per), jnp.float32),
            pltpu.VMEM((k, n_per), jnp.bfloat16),
            pltpu.VMEM((N_DEV, m_half, k), jnp.bfloat16),
            pltpu.VMEM((N_DEV, m_half, k), jnp.bfloat16),
            pltpu.VMEM((2, m_sub, n_per), jnp.float32),
            pltpu.SemaphoreType.DMA((2, N_SUB)),
            pltpu.SemaphoreType.DMA,
            pltpu.SemaphoreType.DMA((n_stores,)),
            pltpu.SemaphoreType.DMA((N_DEV - 1, N_SUB)),
            pltpu.SemaphoreType.DMA((N_DEV - 1, N_SUB)),
            pltpu.SemaphoreType.DMA((N_DEV - 1, N_SUB)),
            pltpu.SemaphoreType.DMA((N_DEV - 1, N_SUB)),
        ],
        compiler_params=pltpu.CompilerParams(collective_id=0),
    )(x, w_mat)
